# baseline (speedup 1.0000x reference)
import functools

import jax
import jax.numpy as jnp
from jax import lax
from jax.experimental import pallas as pl
from jax.experimental.pallas import tpu as pltpu
from jax.experimental.pallas import tpu_sc as plsc

N = 10000
H = 128
E = 320000
NSC = 2
NSUB = 16
NPASS = 2
HQ = H // (NSC * NPASS)
ET = E // NSUB
SCE = 4000
NSUP = ET // SCE
K = 80
NCHS = SCE // K
ZR = 40
ZJOB = 2000


def _pre_body(x_ref, h_ref, c_ref, wf_ref, uf_ref, bf_ref,
              wfx_out, p_out, h_out, c_out):
    xv = x_ref[...]
    hv = h_ref[...]
    uf = uf_ref[...]
    wfx = jnp.dot(xv, wf_ref[...], preferred_element_type=jnp.float32) + bf_ref[...]
    wfx_out[...] = wfx.astype(jnp.bfloat16)
    p0 = jnp.dot(hv, uf[0:H, :], preferred_element_type=jnp.float32)
    p1 = jnp.dot(hv, uf[H:2 * H, :], preferred_element_type=jnp.float32)
    p_out[...] = jnp.stack([p0, p1], axis=0).astype(jnp.bfloat16)
    h_out[...] = hv.astype(jnp.bfloat16)
    c_out[...] = c_ref[...].astype(jnp.bfloat16)


def _post_body(x_ref, wiou_ref, t_ref, u_ref, biou_ref, cagg_ref, h_out, c_out):
    xv = x_ref[...]
    tv = t_ref[...]
    uv = u_ref[...]
    s_blk = jnp.concatenate(
        [tv[cq, t] for t in range(2) for cq in range(4)], axis=1)
    iou = (jnp.dot(xv, wiou_ref[...], preferred_element_type=jnp.float32)
           + jnp.dot(s_blk, uv, preferred_element_type=jnp.float32)
           + biou_ref[...])
    i = jax.nn.sigmoid(iou[:, 0:H])
    o = jax.nn.sigmoid(iou[:, H:2 * H])
    u = jnp.tanh(iou[:, 2 * H:3 * H])
    ca = cagg_ref[...]
    c_agg = jnp.concatenate([ca[cq] for cq in range(4)], axis=1)
    c_new = i * u + c_agg
    h_out[...] = o * jnp.tanh(c_new)
    c_out[...] = c_new


def _sc_edge_body(src_hbm, dst_hbm, el_hbm, p4_hbm, c4_hbm, h4_hbm, wf4_hbm,
                  t_out, c_out,
                  se_s, se_d, se_l,
                  ip0, ip1, isc0, isc1, iw0, iw1, it0, it1, ic0, ic1,
                  fm0, fm1, cs0, cs1, hs0, hs1, wf0, wf1,
                  fc0, fc1, hf0, hf1,
                  zb, tacc, cacc, g0, g1, s0, s1):
    cidx = lax.axis_index("c")
    sidx = lax.axis_index("s")

    ip_, isc_, iw_, it_, ic_ = (ip0, ip1), (isc0, isc1), (iw0, iw1), (it0, it1), (ic0, ic1)
    fm_, cs_, hs_, wf_ = (fm0, fm1), (cs0, cs1), (hs0, hs1), (wf0, wf1)
    fc_, hf_ = (fc0, fc1), (hf0, hf1)
    gsem, ssem = (g0, g1), (s0, s1)

    def _zb_row(r, _):
        for j in range(HQ // 16):
            zb[r, pl.ds(j * 16, 16)] = jnp.zeros((16,), jnp.float32)
        return _
    lax.fori_loop(0, ZR, _zb_row, None)

    for q in range(NPASS):
        cq = 2 * q + cidx

        for w in range(15):
            @pl.when(sidx == w)
            def _zero_job(w=w):
                def _z(i, _):
                    if w < 10:
                        pltpu.sync_copy(zb, tacc.at[pl.ds(w * ZJOB + i * ZR, ZR)])
                    else:
                        pltpu.sync_copy(zb, cacc.at[pl.ds((w - 10) * ZJOB + i * ZR, ZR)])
                    return _
                lax.fori_loop(0, ZJOB // ZR, _z, None)

        plsc.subcore_barrier()

        def _idx(b, o):
            for sl in range(K // 16):
                sls = pl.ds(sl * 16, 16)
                s_v = se_s[pl.ds(o + sl * 16, 16)]
                d_v = se_d[pl.ds(o + sl * 16, 16)]
                l_v = se_l[pl.ds(o + sl * 16, 16)].astype(jnp.int32)
                ip_[b][sls] = 4 * (s_v + N * l_v) + cq
                isc_[b][sls] = 4 * s_v + cq
                iw_[b][sls] = 4 * d_v + cq
                it_[b][sls] = d_v + N * l_v
                ic_[b][sls] = d_v

        def _gather_issue(b):
            pltpu.async_copy(p4_hbm.at[ip_[b]], fm_[b], gsem[b])
            pltpu.async_copy(c4_hbm.at[isc_[b]], cs_[b], gsem[b])
            pltpu.async_copy(h4_hbm.at[isc_[b]], hs_[b], gsem[b])
            pltpu.async_copy(wf4_hbm.at[iw_[b]], wf_[b], gsem[b])

        def _gather_wait(b):
            pltpu.make_async_copy(p4_hbm.at[ip_[b]], fm_[b], gsem[b]).wait()
            pltpu.make_async_copy(c4_hbm.at[isc_[b]], cs_[b], gsem[b]).wait()
            pltpu.make_async_copy(h4_hbm.at[isc_[b]], hs_[b], gsem[b]).wait()
            pltpu.make_async_copy(wf4_hbm.at[iw_[b]], wf_[b], gsem[b]).wait()

        def _scatter_issue(b):
            pltpu.async_copy(fc_[b], cacc.at[ic_[b]], ssem[b], add=True)
            pltpu.async_copy(hf_[b], tacc.at[it_[b]], ssem[b], add=True)

        def _scatter_wait(b):
            pltpu.make_async_copy(fc_[b], cacc.at[ic_[b]], ssem[b]).wait()
            pltpu.make_async_copy(hf_[b], tacc.at[it_[b]], ssem[b]).wait()

        MASK_HI = jnp.int32(-65536)

        def _bf2(v):
            lo = lax.bitcast_convert_type(v << 16, jnp.float32)
            hi = lax.bitcast_convert_type(v & MASK_HI, jnp.float32)
            return lo, hi

        def _compute_f(b):
            def _f_row(e4, _3):
                for r in range(4):
                    e = e4 * 4 + r
                    sl0 = pl.ds(0, 16)
                    wa, wb = _bf2(wf_[b][e, sl0])
                    fa, fb = _bf2(fm_[b][e, sl0])
                    ca, cb = _bf2(cs_[b][e, sl0])
                    ha, hb = _bf2(hs_[b][e, sl0])
                    fva = 1.0 / (1.0 + jnp.exp(-(wa + fa)))
                    fvb = 1.0 / (1.0 + jnp.exp(-(wb + fb)))
                    fc_[b][e, pl.ds(0, 16)] = fva * ca
                    fc_[b][e, pl.ds(16, 16)] = fvb * cb
                    hf_[b][e, pl.ds(0, 16)] = ha
                    hf_[b][e, pl.ds(16, 16)] = hb
                return _3
            lax.fori_loop(0, K // 4, _f_row, None)

        def _sup(s, _):
            base = sidx * ET + s * SCE
            pltpu.sync_copy(src_hbm.at[pl.ds(base, SCE)], se_s)
            pltpu.sync_copy(dst_hbm.at[pl.ds(base, SCE)], se_d)
            pltpu.sync_copy(el_hbm.at[pl.ds(base, SCE)], se_l)

            _idx(0, 0)
            _gather_issue(0)

            def _pair(j2, _2):
                for b in range(2):
                    j = j2 * 2 + b
                    b1 = 1 - b
                    _gather_wait(b)

                    @pl.when(j > 0)
                    def _w():
                        _scatter_wait(b1)

                    @pl.when(j < NCHS - 1)
                    def _n():
                        _idx(b1, (j + 1) * K)
                        _gather_issue(b1)

                    _compute_f(b)
                    _scatter_issue(b)
                return _2
            lax.fori_loop(0, NCHS // 2, _pair, None)
            _scatter_wait(1)
            return _
        lax.fori_loop(0, NSUP, _sup, None)

        plsc.subcore_barrier()

        for w in range(15):
            @pl.when(sidx == w)
            def _out_job(w=w):
                if w < 10:
                    t, i = w // 5, w % 5
                    pltpu.sync_copy(tacc.at[pl.ds(t * N + i * ZJOB, ZJOB)],
                                    t_out.at[cq, t, pl.ds(i * ZJOB, ZJOB)])
                else:
                    i = w - 10
                    pltpu.sync_copy(cacc.at[pl.ds(i * ZJOB, ZJOB)],
                                    c_out.at[cq, pl.ds(i * ZJOB, ZJOB)])

        plsc.subcore_barrier()


@functools.lru_cache(maxsize=1)
def _sc_edge():
  return pl.kernel(
    _sc_edge_body,
    mesh=plsc.VectorSubcoreMesh(core_axis_name="c", subcore_axis_name="s"),
    compiler_params=pltpu.CompilerParams(use_tc_tiling_on_sc=False),
    out_type=[
        jax.ShapeDtypeStruct((4, 2, N, HQ), jnp.float32),
        jax.ShapeDtypeStruct((4, N, HQ), jnp.float32),
    ],
    scratch_types=(
        [pltpu.VMEM((SCE,), jnp.int32),
         pltpu.VMEM((SCE,), jnp.int32),
         pltpu.VMEM((SCE,), jnp.float32)]
        + [pltpu.VMEM((K,), jnp.int32)] * 10
        + [pltpu.VMEM((K, HQ // 2), jnp.int32)] * 8
        + [pltpu.VMEM((K, HQ), jnp.float32)] * 4
        + [pltpu.VMEM((ZR, HQ), jnp.float32),
           pltpu.VMEM_SHARED((2 * N, HQ), jnp.float32),
           pltpu.VMEM_SHARED((N, HQ), jnp.float32)]
        + [pltpu.SemaphoreType.DMA] * 4
    ),
  )


def kernel(x, h, c, edge_index, edge_label, W_iou, W_f, U_iou, U_f, b_iou, b_f):
    bm = 2000
    grid = (N // bm,)

    wf_x, p, h_bf, c_bf = pl.pallas_call(
        _pre_body,
        grid=grid,
        in_specs=[
            pl.BlockSpec((bm, H), lambda i: (i, 0)),
            pl.BlockSpec((bm, H), lambda i: (i, 0)),
            pl.BlockSpec((bm, H), lambda i: (i, 0)),
            pl.BlockSpec((H, H), lambda i: (0, 0)),
            pl.BlockSpec((2 * H, H), lambda i: (0, 0)),
            pl.BlockSpec((1, H), lambda i: (0, 0)),
        ],
        out_specs=[
            pl.BlockSpec((bm, H), lambda i: (i, 0)),
            pl.BlockSpec((2, bm, H), lambda i: (0, i, 0)),
            pl.BlockSpec((bm, H), lambda i: (i, 0)),
            pl.BlockSpec((bm, H), lambda i: (i, 0)),
        ],
        out_shape=[
            jax.ShapeDtypeStruct((N, H), jnp.bfloat16),
            jax.ShapeDtypeStruct((2, N, H), jnp.bfloat16),
            jax.ShapeDtypeStruct((N, H), jnp.bfloat16),
            jax.ShapeDtypeStruct((N, H), jnp.bfloat16),
        ],
    )(x, h, c, W_f, U_f, b_f)

    src = edge_index[0]
    dst = edge_index[1]
    el = edge_label[:, 0]

    def _tbl(a):
        xdim = a.shape[0]
        q = jnp.swapaxes(a.reshape(xdim, 4, 2, 16), 2, 3)
        return lax.bitcast_convert_type(q, jnp.int32).reshape(4 * xdim, HQ // 2)

    p4 = _tbl(p.reshape(2 * N, H))
    wf4 = _tbl(wf_x)
    h4 = _tbl(h_bf)
    c4 = _tbl(c_bf)

    t_acc, c_agg = _sc_edge()(src, dst, el, p4, c4, h4, wf4)

    h_new, c_new = pl.pallas_call(
        _post_body,
        grid=grid,
        in_specs=[
            pl.BlockSpec((bm, H), lambda i: (i, 0)),
            pl.BlockSpec((H, 3 * H), lambda i: (0, 0)),
            pl.BlockSpec((4, 2, bm, HQ), lambda i: (0, 0, i, 0)),
            pl.BlockSpec((2 * H, 3 * H), lambda i: (0, 0)),
            pl.BlockSpec((1, 3 * H), lambda i: (0, 0)),
            pl.BlockSpec((4, bm, HQ), lambda i: (0, i, 0)),
        ],
        out_specs=[
            pl.BlockSpec((bm, H), lambda i: (i, 0)),
            pl.BlockSpec((bm, H), lambda i: (i, 0)),
        ],
        out_shape=[
            jax.ShapeDtypeStruct((N, H), jnp.float32),
            jax.ShapeDtypeStruct((N, H), jnp.float32),
        ],
    )(x, W_iou, t_acc, U_iou, b_iou, c_agg)

    return h_new, c_new


# speedup vs baseline: 1.4110x; 1.4110x over previous
import functools

import jax
import jax.numpy as jnp
from jax import lax
from jax.experimental import pallas as pl
from jax.experimental.pallas import tpu as pltpu
from jax.experimental.pallas import tpu_sc as plsc

N = 10000
H = 128
E = 320000
NSC = 2
NSUB = 16
NPASS = 2
HQ = H // (NSC * NPASS)
ET = E // NSUB
SCE = 4000
NSUP = ET // SCE
K = 80
NCHS = SCE // K
ZR = 40
ZJOB = 2000


def _pre_body(x_ref, h_ref, wf_ref, uf_ref, bf_ref, wfx_out, p_out):
    xv = x_ref[...]
    hv = h_ref[...]
    uf = uf_ref[...]
    wfx_out[...] = jnp.dot(xv, wf_ref[...], preferred_element_type=jnp.float32) + bf_ref[...]
    p0 = jnp.dot(hv, uf[0:H, :], preferred_element_type=jnp.float32)
    p1 = jnp.dot(hv, uf[H:2 * H, :], preferred_element_type=jnp.float32)
    p_out[...] = jnp.stack([p0, p1], axis=0)


def _post_body(x_ref, wiou_ref, t_ref, u_ref, biou_ref, cagg_ref, h_out, c_out):
    xv = x_ref[...]
    tv = t_ref[...]
    uv = u_ref[...]
    s_blk = jnp.concatenate(
        [tv[cq, t] for t in range(2) for cq in range(4)], axis=1)
    iou = (jnp.dot(xv, wiou_ref[...], preferred_element_type=jnp.float32)
           + jnp.dot(s_blk, uv, preferred_element_type=jnp.float32)
           + biou_ref[...])
    i = jax.nn.sigmoid(iou[:, 0:H])
    o = jax.nn.sigmoid(iou[:, H:2 * H])
    u = jnp.tanh(iou[:, 2 * H:3 * H])
    ca = cagg_ref[...]
    c_agg = jnp.concatenate([ca[cq] for cq in range(4)], axis=1)
    c_new = i * u + c_agg
    h_out[...] = o * jnp.tanh(c_new)
    c_out[...] = c_new


def _sc_edge_body(src_hbm, dst_hbm, el_hbm, p4_hbm, hc4_hbm, wf4_hbm,
                  t_out, c_out,
                  se_s, se_d, se_l,
                  ip0, ip1, isc0, isc1, iw0, iw1, it0, it1, ic0, ic1,
                  fm0, fm1, hcs0, hcs1, wf0, wf1, fc0, fc1, hf0, hf1,
                  zb, tacc, cacc, g0, g1, s0, s1):
    cidx = lax.axis_index("c")
    sidx = lax.axis_index("s")

    ip_, isc_, iw_, it_, ic_ = (ip0, ip1), (isc0, isc1), (iw0, iw1), (it0, it1), (ic0, ic1)
    fm_, hcs_, wf_ = (fm0, fm1), (hcs0, hcs1), (wf0, wf1)
    fc_, hf_ = (fc0, fc1), (hf0, hf1)
    gsem, ssem = (g0, g1), (s0, s1)

    def _zb_row(r, _):
        for j in range(HQ // 16):
            zb[r, pl.ds(j * 16, 16)] = jnp.zeros((16,), jnp.float32)
        return _
    lax.fori_loop(0, ZR, _zb_row, None)

    for q in range(NPASS):
        cq = 2 * q + cidx

        for w in range(15):
            @pl.when(sidx == w)
            def _zero_job(w=w):
                def _z(i, _):
                    if w < 10:
                        pltpu.sync_copy(zb, tacc.at[pl.ds(w * ZJOB + i * ZR, ZR)])
                    else:
                        pltpu.sync_copy(zb, cacc.at[pl.ds((w - 10) * ZJOB + i * ZR, ZR)])
                    return _
                lax.fori_loop(0, ZJOB // ZR, _z, None)

        plsc.subcore_barrier()

        def _idx(b, o):
            for sl in range(K // 16):
                sls = pl.ds(sl * 16, 16)
                s_v = se_s[pl.ds(o + sl * 16, 16)]
                d_v = se_d[pl.ds(o + sl * 16, 16)]
                l_v = se_l[pl.ds(o + sl * 16, 16)].astype(jnp.int32)
                ip_[b][sls] = 4 * (s_v + N * l_v) + cq
                isc_[b][sls] = 4 * s_v + cq
                iw_[b][sls] = 4 * d_v + cq
                it_[b][sls] = d_v + N * l_v
                ic_[b][sls] = d_v

        def _gather_issue(b):
            pltpu.async_copy(p4_hbm.at[ip_[b]], fm_[b], gsem[b])
            pltpu.async_copy(hc4_hbm.at[isc_[b]], hcs_[b], gsem[b])
            pltpu.async_copy(wf4_hbm.at[iw_[b]], wf_[b], gsem[b])

        def _gather_wait(b):
            pltpu.make_async_copy(p4_hbm.at[ip_[b]], fm_[b], gsem[b]).wait()
            pltpu.make_async_copy(hc4_hbm.at[isc_[b]], hcs_[b], gsem[b]).wait()
            pltpu.make_async_copy(wf4_hbm.at[iw_[b]], wf_[b], gsem[b]).wait()

        def _scatter_issue(b):
            pltpu.async_copy(fc_[b], cacc.at[ic_[b]], ssem[b], add=True)
            pltpu.async_copy(hf_[b], tacc.at[it_[b]], ssem[b], add=True)

        def _scatter_wait(b):
            pltpu.make_async_copy(fc_[b], cacc.at[ic_[b]], ssem[b]).wait()
            pltpu.make_async_copy(hf_[b], tacc.at[it_[b]], ssem[b]).wait()

        def _compute_f(b):
            def _f_row(e4, _3):
                for r in range(4):
                    e = e4 * 4 + r
                    for jj in range(HQ // 16):
                        sl_ = pl.ds(jj * 16, 16)
                        xv = wf_[b][e, sl_] + fm_[b][e, sl_]
                        fv = 1.0 / (1.0 + jnp.exp(-xv))
                        fc_[b][e, sl_] = fv * hcs_[b][e, pl.ds(HQ + jj * 16, 16)]
                        hf_[b][e, sl_] = hcs_[b][e, pl.ds(jj * 16, 16)]
                return _3
            lax.fori_loop(0, K // 4, _f_row, None)

        def _sup(s, _):
            base = sidx * ET + s * SCE
            pltpu.sync_copy(src_hbm.at[pl.ds(base, SCE)], se_s)
            pltpu.sync_copy(dst_hbm.at[pl.ds(base, SCE)], se_d)
            pltpu.sync_copy(el_hbm.at[pl.ds(base, SCE)], se_l)

            _idx(0, 0)
            _gather_issue(0)

            def _pair(j2, _2):
                for b in range(2):
                    j = j2 * 2 + b
                    b1 = 1 - b
                    _gather_wait(b)

                    @pl.when(j > 0)
                    def _w():
                        _scatter_wait(b1)

                    @pl.when(j < NCHS - 1)
                    def _n():
                        _idx(b1, (j + 1) * K)
                        _gather_issue(b1)

                    _compute_f(b)
                    _scatter_issue(b)
                return _2
            lax.fori_loop(0, NCHS // 2, _pair, None)
            _scatter_wait(1)
            return _
        lax.fori_loop(0, NSUP, _sup, None)

        plsc.subcore_barrier()

        for w in range(15):
            @pl.when(sidx == w)
            def _out_job(w=w):
                if w < 10:
                    t, i = w // 5, w % 5
                    pltpu.sync_copy(tacc.at[pl.ds(t * N + i * ZJOB, ZJOB)],
                                    t_out.at[cq, t, pl.ds(i * ZJOB, ZJOB)])
                else:
                    i = w - 10
                    pltpu.sync_copy(cacc.at[pl.ds(i * ZJOB, ZJOB)],
                                    c_out.at[cq, pl.ds(i * ZJOB, ZJOB)])

        plsc.subcore_barrier()


@functools.lru_cache(maxsize=1)
def _sc_edge():
  return pl.kernel(
    _sc_edge_body,
    mesh=plsc.VectorSubcoreMesh(core_axis_name="c", subcore_axis_name="s"),
    compiler_params=pltpu.CompilerParams(use_tc_tiling_on_sc=False),
    out_type=[
        jax.ShapeDtypeStruct((4, 2, N, HQ), jnp.float32),
        jax.ShapeDtypeStruct((4, N, HQ), jnp.float32),
    ],
    scratch_types=(
        [pltpu.VMEM((SCE,), jnp.int32),
         pltpu.VMEM((SCE,), jnp.int32),
         pltpu.VMEM((SCE,), jnp.float32)]
        + [pltpu.VMEM((K,), jnp.int32)] * 10
        + [pltpu.VMEM((K, HQ), jnp.float32)] * 2
        + [pltpu.VMEM((K, 2 * HQ), jnp.float32)] * 2
        + [pltpu.VMEM((K, HQ), jnp.float32)] * 6
        + [pltpu.VMEM((ZR, HQ), jnp.float32),
           pltpu.VMEM_SHARED((2 * N, HQ), jnp.float32),
           pltpu.VMEM_SHARED((N, HQ), jnp.float32)]
        + [pltpu.SemaphoreType.DMA] * 4
    ),
  )


def kernel(x, h, c, edge_index, edge_label, W_iou, W_f, U_iou, U_f, b_iou, b_f):
    bm = 2000
    grid = (N // bm,)

    wf_x, p = pl.pallas_call(
        _pre_body,
        grid=grid,
        in_specs=[
            pl.BlockSpec((bm, H), lambda i: (i, 0)),
            pl.BlockSpec((bm, H), lambda i: (i, 0)),
            pl.BlockSpec((H, H), lambda i: (0, 0)),
            pl.BlockSpec((2 * H, H), lambda i: (0, 0)),
            pl.BlockSpec((1, H), lambda i: (0, 0)),
        ],
        out_specs=[
            pl.BlockSpec((bm, H), lambda i: (i, 0)),
            pl.BlockSpec((2, bm, H), lambda i: (0, i, 0)),
        ],
        out_shape=[
            jax.ShapeDtypeStruct((N, H), jnp.float32),
            jax.ShapeDtypeStruct((2, N, H), jnp.float32),
        ],
    )(x, h, W_f, U_f, b_f)

    src = edge_index[0]
    dst = edge_index[1]
    el = edge_label[:, 0]
    p4 = p.reshape(2 * N, 4, HQ).reshape(8 * N, HQ)
    wf4 = wf_x.reshape(4 * N, HQ)
    hc4 = jnp.concatenate(
        [h.reshape(N, 4, HQ), c.reshape(N, 4, HQ)], axis=2).reshape(4 * N, 2 * HQ)

    t_acc, c_agg = _sc_edge()(src, dst, el, p4, hc4, wf4)

    h_new, c_new = pl.pallas_call(
        _post_body,
        grid=grid,
        in_specs=[
            pl.BlockSpec((bm, H), lambda i: (i, 0)),
            pl.BlockSpec((H, 3 * H), lambda i: (0, 0)),
            pl.BlockSpec((4, 2, bm, HQ), lambda i: (0, 0, i, 0)),
            pl.BlockSpec((2 * H, 3 * H), lambda i: (0, 0)),
            pl.BlockSpec((1, 3 * H), lambda i: (0, 0)),
            pl.BlockSpec((4, bm, HQ), lambda i: (0, i, 0)),
        ],
        out_specs=[
            pl.BlockSpec((bm, H), lambda i: (i, 0)),
            pl.BlockSpec((bm, H), lambda i: (i, 0)),
        ],
        out_shape=[
            jax.ShapeDtypeStruct((N, H), jnp.float32),
            jax.ShapeDtypeStruct((N, H), jnp.float32),
        ],
    )(x, W_iou, t_acc, U_iou, b_iou, c_agg)

    return h_new, c_new


# speedup vs baseline: 2.0427x; 1.4477x over previous
import functools

import jax
import jax.numpy as jnp
from jax import lax
from jax.experimental import pallas as pl
from jax.experimental.pallas import tpu as pltpu
from jax.experimental.pallas import tpu_sc as plsc

N = 10000
H = 128
E = 320000
NSC = 2
NSUB = 16
NPASS = 2
HQ = H // (NSC * NPASS)
ET = E // NSUB
SCE = 4000
NSUP = ET // SCE
K = 80
NCHS = SCE // K
ZR = 40
ZJOB = 2000


def _pre_body(x_ref, h_ref, wf_ref, uf_ref, bf_ref, wfx_out, p_out):
    xv = x_ref[...]
    hv = h_ref[...]
    uf = uf_ref[...]
    wfx_out[...] = jnp.dot(xv, wf_ref[...], preferred_element_type=jnp.float32) + bf_ref[...]
    p0 = jnp.dot(hv, uf[0:H, :], preferred_element_type=jnp.float32)
    p1 = jnp.dot(hv, uf[H:2 * H, :], preferred_element_type=jnp.float32)
    p_out[...] = jnp.stack([p0, p1], axis=0)


def _post_body(x_ref, wiou_ref, t_ref, u_ref, biou_ref, cagg_ref, h_out, c_out):
    xv = x_ref[...]
    tv = t_ref[...]
    uv = u_ref[...]
    s_blk = jnp.concatenate(
        [tv[cq, t] for t in range(2) for cq in range(4)], axis=1)
    iou = (jnp.dot(xv, wiou_ref[...], preferred_element_type=jnp.float32)
           + jnp.dot(s_blk, uv, preferred_element_type=jnp.float32)
           + biou_ref[...])
    i = jax.nn.sigmoid(iou[:, 0:H])
    o = jax.nn.sigmoid(iou[:, H:2 * H])
    u = jnp.tanh(iou[:, 2 * H:3 * H])
    ca = cagg_ref[...]
    c_agg = jnp.concatenate([ca[cq] for cq in range(4)], axis=1)
    c_new = i * u + c_agg
    h_out[...] = o * jnp.tanh(c_new)
    c_out[...] = c_new


def _sc_edge_body(src_hbm, dst_hbm, el_hbm, p4_hbm, hc4_hbm, wf4_hbm,
                  t_out, c_out,
                  se_s, se_d, se_l,
                  ip0, ip1, isc0, isc1, iw0, iw1, it0, it1, ic0, ic1,
                  fm0, fm1, hcs0, hcs1, wf0, wf1, fc0, fc1, hf0, hf1,
                  zb, tacc, cacc, g0, g1, s0, s1):
    cidx = lax.axis_index("c")
    sidx = lax.axis_index("s")

    ip_, isc_, iw_, it_, ic_ = (ip0, ip1), (isc0, isc1), (iw0, iw1), (it0, it1), (ic0, ic1)
    fm_, hcs_, wf_ = (fm0, fm1), (hcs0, hcs1), (wf0, wf1)
    fc_, hf_ = (fc0, fc1), (hf0, hf1)
    gsem, ssem = (g0, g1), (s0, s1)

    def _zb_row(r, _):
        for j in range(HQ // 16):
            zb[r, pl.ds(j * 16, 16)] = jnp.zeros((16,), jnp.float32)
        return _
    lax.fori_loop(0, ZR, _zb_row, None)

    for q in range(NPASS):
        cq = 2 * q + cidx

        for w in range(15):
            @pl.when(sidx == w)
            def _zero_job(w=w):
                def _z(i, _):
                    if w < 10:
                        pltpu.sync_copy(zb, tacc.at[pl.ds(w * ZJOB + i * ZR, ZR)])
                    else:
                        pltpu.sync_copy(zb, cacc.at[pl.ds((w - 10) * ZJOB + i * ZR, ZR)])
                    return _
                lax.fori_loop(0, ZJOB // ZR, _z, None)

        plsc.subcore_barrier()

        def _idx(b, o):
            for sl in range(K // 16):
                sls = pl.ds(sl * 16, 16)
                s_v = se_s[pl.ds(o + sl * 16, 16)]
                d_v = se_d[pl.ds(o + sl * 16, 16)]
                l_v = se_l[pl.ds(o + sl * 16, 16)].astype(jnp.int32)
                ip_[b][sls] = 4 * (s_v + N * l_v) + cq
                isc_[b][sls] = 4 * s_v + cq
                iw_[b][sls] = 4 * d_v + cq
                it_[b][sls] = d_v + N * l_v
                ic_[b][sls] = d_v

        def _gather_issue(b):
            pltpu.async_copy(p4_hbm.at[ip_[b]], fm_[b], gsem[b])
            pltpu.async_copy(hc4_hbm.at[isc_[b]], hcs_[b], gsem[b])
            pltpu.async_copy(wf4_hbm.at[iw_[b]], wf_[b], gsem[b])

        def _gather_wait(b):
            pltpu.make_async_copy(p4_hbm.at[ip_[b]], fm_[b], gsem[b]).wait()
            pltpu.make_async_copy(hc4_hbm.at[isc_[b]], hcs_[b], gsem[b]).wait()
            pltpu.make_async_copy(wf4_hbm.at[iw_[b]], wf_[b], gsem[b]).wait()

        def _scatter_issue(b):
            pltpu.async_copy(fc_[b], cacc.at[ic_[b]], ssem[b], add=True)
            pltpu.async_copy(hf_[b], tacc.at[it_[b]], ssem[b], add=True)

        def _scatter_wait(b):
            pltpu.make_async_copy(fc_[b], cacc.at[ic_[b]], ssem[b]).wait()
            pltpu.make_async_copy(hf_[b], tacc.at[it_[b]], ssem[b]).wait()

        MASK_HI = jnp.int32(-65536)

        def _bf2(v):
            lo = lax.bitcast_convert_type(v << 16, jnp.float32)
            hi = lax.bitcast_convert_type(v & MASK_HI, jnp.float32)
            return lo, hi

        def _compute_f(b):
            def _f_row(e4, _3):
                for r in range(4):
                    e = e4 * 4 + r
                    ha, hb = _bf2(hcs_[b][e, pl.ds(0, 16)])
                    ca, cb = _bf2(hcs_[b][e, pl.ds(16, 16)])
                    xa = wf_[b][e, pl.ds(0, 16)] + fm_[b][e, pl.ds(0, 16)]
                    xb = wf_[b][e, pl.ds(16, 16)] + fm_[b][e, pl.ds(16, 16)]
                    fva = 1.0 / (1.0 + jnp.exp(-xa))
                    fvb = 1.0 / (1.0 + jnp.exp(-xb))
                    fc_[b][e, pl.ds(0, 16)] = fva * ca
                    fc_[b][e, pl.ds(16, 16)] = fvb * cb
                    hf_[b][e, pl.ds(0, 16)] = ha
                    hf_[b][e, pl.ds(16, 16)] = hb
                return _3
            lax.fori_loop(0, K // 4, _f_row, None)

        def _sup(s, _):
            base = sidx * ET + s * SCE
            pltpu.sync_copy(src_hbm.at[pl.ds(base, SCE)], se_s)
            pltpu.sync_copy(dst_hbm.at[pl.ds(base, SCE)], se_d)
            pltpu.sync_copy(el_hbm.at[pl.ds(base, SCE)], se_l)

            _idx(0, 0)
            _gather_issue(0)

            def _pair(j2, _2):
                for b in range(2):
                    j = j2 * 2 + b
                    b1 = 1 - b
                    _gather_wait(b)

                    @pl.when(j > 0)
                    def _w():
                        _scatter_wait(b1)

                    @pl.when(j < NCHS - 1)
                    def _n():
                        _idx(b1, (j + 1) * K)
                        _gather_issue(b1)

                    _compute_f(b)
                    _scatter_issue(b)
                return _2
            lax.fori_loop(0, NCHS // 2, _pair, None)
            _scatter_wait(1)
            return _
        lax.fori_loop(0, NSUP, _sup, None)

        plsc.subcore_barrier()

        for w in range(15):
            @pl.when(sidx == w)
            def _out_job(w=w):
                if w < 10:
                    t, i = w // 5, w % 5
                    pltpu.sync_copy(tacc.at[pl.ds(t * N + i * ZJOB, ZJOB)],
                                    t_out.at[cq, t, pl.ds(i * ZJOB, ZJOB)])
                else:
                    i = w - 10
                    pltpu.sync_copy(cacc.at[pl.ds(i * ZJOB, ZJOB)],
                                    c_out.at[cq, pl.ds(i * ZJOB, ZJOB)])

        plsc.subcore_barrier()


@functools.lru_cache(maxsize=1)
def _sc_edge():
  return pl.kernel(
    _sc_edge_body,
    mesh=plsc.VectorSubcoreMesh(core_axis_name="c", subcore_axis_name="s"),
    compiler_params=pltpu.CompilerParams(use_tc_tiling_on_sc=False),
    out_type=[
        jax.ShapeDtypeStruct((4, 2, N, HQ), jnp.float32),
        jax.ShapeDtypeStruct((4, N, HQ), jnp.float32),
    ],
    scratch_types=(
        [pltpu.VMEM((SCE,), jnp.int32),
         pltpu.VMEM((SCE,), jnp.int32),
         pltpu.VMEM((SCE,), jnp.float32)]
        + [pltpu.VMEM((K,), jnp.int32)] * 10
        + [pltpu.VMEM((K, HQ), jnp.float32)] * 2
        + [pltpu.VMEM((K, HQ), jnp.int32)] * 2
        + [pltpu.VMEM((K, HQ), jnp.float32)] * 6
        + [pltpu.VMEM((ZR, HQ), jnp.float32),
           pltpu.VMEM_SHARED((2 * N, HQ), jnp.float32),
           pltpu.VMEM_SHARED((N, HQ), jnp.float32)]
        + [pltpu.SemaphoreType.DMA] * 4
    ),
  )


def kernel(x, h, c, edge_index, edge_label, W_iou, W_f, U_iou, U_f, b_iou, b_f):
    bm = 2000
    grid = (N // bm,)

    wf_x, p = pl.pallas_call(
        _pre_body,
        grid=grid,
        in_specs=[
            pl.BlockSpec((bm, H), lambda i: (i, 0)),
            pl.BlockSpec((bm, H), lambda i: (i, 0)),
            pl.BlockSpec((H, H), lambda i: (0, 0)),
            pl.BlockSpec((2 * H, H), lambda i: (0, 0)),
            pl.BlockSpec((1, H), lambda i: (0, 0)),
        ],
        out_specs=[
            pl.BlockSpec((bm, H), lambda i: (i, 0)),
            pl.BlockSpec((2, bm, H), lambda i: (0, i, 0)),
        ],
        out_shape=[
            jax.ShapeDtypeStruct((N, H), jnp.float32),
            jax.ShapeDtypeStruct((2, N, H), jnp.float32),
        ],
    )(x, h, W_f, U_f, b_f)

    src = edge_index[0]
    dst = edge_index[1]
    el = edge_label[:, 0]
    p4 = p.reshape(2 * N, 4, HQ).reshape(8 * N, HQ)
    wf4 = wf_x.reshape(4 * N, HQ)
    def _pairs(a):
        return jnp.swapaxes(a.astype(jnp.bfloat16).reshape(N, 4, 2, 16), 2, 3)
    hc4 = lax.bitcast_convert_type(
        jnp.concatenate([_pairs(h), _pairs(c)], axis=2), jnp.int32
    ).reshape(4 * N, HQ)

    t_acc, c_agg = _sc_edge()(src, dst, el, p4, hc4, wf4)

    h_new, c_new = pl.pallas_call(
        _post_body,
        grid=grid,
        in_specs=[
            pl.BlockSpec((bm, H), lambda i: (i, 0)),
            pl.BlockSpec((H, 3 * H), lambda i: (0, 0)),
            pl.BlockSpec((4, 2, bm, HQ), lambda i: (0, 0, i, 0)),
            pl.BlockSpec((2 * H, 3 * H), lambda i: (0, 0)),
            pl.BlockSpec((1, 3 * H), lambda i: (0, 0)),
            pl.BlockSpec((4, bm, HQ), lambda i: (0, i, 0)),
        ],
        out_specs=[
            pl.BlockSpec((bm, H), lambda i: (i, 0)),
            pl.BlockSpec((bm, H), lambda i: (i, 0)),
        ],
        out_shape=[
            jax.ShapeDtypeStruct((N, H), jnp.float32),
            jax.ShapeDtypeStruct((N, H), jnp.float32),
        ],
    )(x, W_iou, t_acc, U_iou, b_iou, c_agg)

    return h_new, c_new


# speedup vs baseline: 2.0441x; 1.0006x over previous
import functools

import jax
import jax.numpy as jnp
from jax import lax
from jax.experimental import pallas as pl
from jax.experimental.pallas import tpu as pltpu
from jax.experimental.pallas import tpu_sc as plsc

N = 10000
H = 128
E = 320000
NSC = 2
NSUB = 16
NPASS = 2
HQ = H // (NSC * NPASS)
ET = E // NSUB
SCE = 4000
NSUP = ET // SCE
K = 80
NCHS = SCE // K
ZR = 40
ZJOB = 2000


def _pre_body(x_ref, h_ref, wf_ref, uf_ref, bf_ref, wfx_out, p_out):
    xv = x_ref[...]
    hv = h_ref[...]
    uf = uf_ref[...]
    wfx_out[...] = jnp.dot(xv, wf_ref[...], preferred_element_type=jnp.float32) + bf_ref[...]
    p0 = jnp.dot(hv, uf[0:H, :], preferred_element_type=jnp.float32)
    p1 = jnp.dot(hv, uf[H:2 * H, :], preferred_element_type=jnp.float32)
    p_out[...] = jnp.stack([p0, p1], axis=0)


def _post_body(x_ref, wiou_ref, t_ref, u_ref, biou_ref, cagg_ref, h_out, c_out):
    xv = x_ref[...]
    tv = t_ref[...]
    uv = u_ref[...]
    s_blk = jnp.concatenate(
        [tv[cq, t] for t in range(2) for cq in range(4)], axis=1)
    iou = (jnp.dot(xv, wiou_ref[...], preferred_element_type=jnp.float32)
           + jnp.dot(s_blk, uv, preferred_element_type=jnp.float32)
           + biou_ref[...])
    i = jax.nn.sigmoid(iou[:, 0:H])
    o = jax.nn.sigmoid(iou[:, H:2 * H])
    u = jnp.tanh(iou[:, 2 * H:3 * H])
    ca = cagg_ref[...]
    c_agg = jnp.concatenate([ca[cq] for cq in range(4)], axis=1)
    c_new = i * u + c_agg
    h_out[...] = o * jnp.tanh(c_new)
    c_out[...] = c_new


def _sc_edge_body(src_hbm, dst_hbm, el_hbm, p4_hbm, hc4_hbm, wf4_hbm,
                  t_out, c_out,
                  se_s, se_d, se_l,
                  ip0, ip1, isc0, isc1, iw0, iw1, it0, it1, ic0, ic1,
                  fm0, fm1, hcs0, hcs1, wf0, wf1, fc0, fc1, hf0, hf1,
                  zb, tacc, cacc, g0, g1, s0, s1):
    cidx = lax.axis_index("c")
    sidx = lax.axis_index("s")

    ip_, isc_, iw_, it_, ic_ = (ip0, ip1), (isc0, isc1), (iw0, iw1), (it0, it1), (ic0, ic1)
    fm_, hcs_, wf_ = (fm0, fm1), (hcs0, hcs1), (wf0, wf1)
    fc_, hf_ = (fc0, fc1), (hf0, hf1)
    gsem, ssem = (g0, g1), (s0, s1)

    def _zb_row(r, _):
        for j in range(HQ // 16):
            zb[r, pl.ds(j * 16, 16)] = jnp.zeros((16,), jnp.float32)
        return _
    lax.fori_loop(0, ZR, _zb_row, None)

    for q in range(NPASS):
        cq = 2 * q + cidx

        for w in range(15):
            @pl.when(sidx == w)
            def _zero_job(w=w):
                def _z(i, _):
                    if w < 10:
                        pltpu.sync_copy(zb, tacc.at[pl.ds(w * ZJOB + i * ZR, ZR)])
                    else:
                        pltpu.sync_copy(zb, cacc.at[pl.ds((w - 10) * ZJOB + i * ZR, ZR)])
                    return _
                lax.fori_loop(0, ZJOB // ZR, _z, None)

        plsc.subcore_barrier()

        def _idx(b, o):
            for sl in range(K // 16):
                sls = pl.ds(sl * 16, 16)
                s_v = se_s[pl.ds(o + sl * 16, 16)]
                d_v = se_d[pl.ds(o + sl * 16, 16)]
                l_v = se_l[pl.ds(o + sl * 16, 16)].astype(jnp.int32)
                ip_[b][sls] = 4 * (s_v + N * l_v) + cq
                isc_[b][sls] = 4 * s_v + cq
                iw_[b][sls] = 4 * d_v + cq
                it_[b][sls] = d_v + N * l_v
                ic_[b][sls] = d_v

        KH = K // 2

        def _gather_issue(b):
            for hh in range(2):
                sh = pl.ds(hh * KH, KH)
                pltpu.async_copy(p4_hbm.at[ip_[b].at[sh]], fm_[b].at[sh], gsem[b])
                pltpu.async_copy(hc4_hbm.at[isc_[b].at[sh]], hcs_[b].at[sh], gsem[b])
                pltpu.async_copy(wf4_hbm.at[iw_[b].at[sh]], wf_[b].at[sh], gsem[b])

        def _gather_wait(b):
            for hh in range(2):
                sh = pl.ds(hh * KH, KH)
                pltpu.make_async_copy(p4_hbm.at[ip_[b].at[sh]], fm_[b].at[sh], gsem[b]).wait()
                pltpu.make_async_copy(hc4_hbm.at[isc_[b].at[sh]], hcs_[b].at[sh], gsem[b]).wait()
                pltpu.make_async_copy(wf4_hbm.at[iw_[b].at[sh]], wf_[b].at[sh], gsem[b]).wait()

        def _scatter_issue(b):
            pltpu.async_copy(fc_[b], cacc.at[ic_[b]], ssem[b], add=True)
            pltpu.async_copy(hf_[b], tacc.at[it_[b]], ssem[b], add=True)

        def _scatter_wait(b):
            pltpu.make_async_copy(fc_[b], cacc.at[ic_[b]], ssem[b]).wait()
            pltpu.make_async_copy(hf_[b], tacc.at[it_[b]], ssem[b]).wait()

        MASK_HI = jnp.int32(-65536)

        def _bf2(v):
            lo = lax.bitcast_convert_type(v << 16, jnp.float32)
            hi = lax.bitcast_convert_type(v & MASK_HI, jnp.float32)
            return lo, hi

        def _compute_f(b):
            def _f_row(e4, _3):
                for r in range(4):
                    e = e4 * 4 + r
                    ha, hb = _bf2(hcs_[b][e, pl.ds(0, 16)])
                    ca, cb = _bf2(hcs_[b][e, pl.ds(16, 16)])
                    xa = wf_[b][e, pl.ds(0, 16)] + fm_[b][e, pl.ds(0, 16)]
                    xb = wf_[b][e, pl.ds(16, 16)] + fm_[b][e, pl.ds(16, 16)]
                    fva = 1.0 / (1.0 + jnp.exp(-xa))
                    fvb = 1.0 / (1.0 + jnp.exp(-xb))
                    fc_[b][e, pl.ds(0, 16)] = fva * ca
                    fc_[b][e, pl.ds(16, 16)] = fvb * cb
                    hf_[b][e, pl.ds(0, 16)] = ha
                    hf_[b][e, pl.ds(16, 16)] = hb
                return _3
            lax.fori_loop(0, K // 4, _f_row, None)

        def _sup(s, _):
            base = sidx * ET + s * SCE
            pltpu.sync_copy(src_hbm.at[pl.ds(base, SCE)], se_s)
            pltpu.sync_copy(dst_hbm.at[pl.ds(base, SCE)], se_d)
            pltpu.sync_copy(el_hbm.at[pl.ds(base, SCE)], se_l)

            _idx(0, 0)
            _gather_issue(0)

            def _pair(j2, _2):
                for b in range(2):
                    j = j2 * 2 + b
                    b1 = 1 - b
                    _gather_wait(b)

                    @pl.when(j > 0)
                    def _w():
                        _scatter_wait(b1)

                    @pl.when(j < NCHS - 1)
                    def _n():
                        _idx(b1, (j + 1) * K)
                        _gather_issue(b1)

                    _compute_f(b)
                    _scatter_issue(b)
                return _2
            lax.fori_loop(0, NCHS // 2, _pair, None)
            _scatter_wait(1)
            return _
        lax.fori_loop(0, NSUP, _sup, None)

        plsc.subcore_barrier()

        for w in range(15):
            @pl.when(sidx == w)
            def _out_job(w=w):
                if w < 10:
                    t, i = w // 5, w % 5
                    pltpu.sync_copy(tacc.at[pl.ds(t * N + i * ZJOB, ZJOB)],
                                    t_out.at[cq, t, pl.ds(i * ZJOB, ZJOB)])
                else:
                    i = w - 10
                    pltpu.sync_copy(cacc.at[pl.ds(i * ZJOB, ZJOB)],
                                    c_out.at[cq, pl.ds(i * ZJOB, ZJOB)])

        plsc.subcore_barrier()


@functools.lru_cache(maxsize=1)
def _sc_edge():
  return pl.kernel(
    _sc_edge_body,
    mesh=plsc.VectorSubcoreMesh(core_axis_name="c", subcore_axis_name="s"),
    compiler_params=pltpu.CompilerParams(use_tc_tiling_on_sc=False),
    out_type=[
        jax.ShapeDtypeStruct((4, 2, N, HQ), jnp.float32),
        jax.ShapeDtypeStruct((4, N, HQ), jnp.float32),
    ],
    scratch_types=(
        [pltpu.VMEM((SCE,), jnp.int32),
         pltpu.VMEM((SCE,), jnp.int32),
         pltpu.VMEM((SCE,), jnp.float32)]
        + [pltpu.VMEM((K,), jnp.int32)] * 10
        + [pltpu.VMEM((K, HQ), jnp.float32)] * 2
        + [pltpu.VMEM((K, HQ), jnp.int32)] * 2
        + [pltpu.VMEM((K, HQ), jnp.float32)] * 6
        + [pltpu.VMEM((ZR, HQ), jnp.float32),
           pltpu.VMEM_SHARED((2 * N, HQ), jnp.float32),
           pltpu.VMEM_SHARED((N, HQ), jnp.float32)]
        + [pltpu.SemaphoreType.DMA] * 4
    ),
  )


def kernel(x, h, c, edge_index, edge_label, W_iou, W_f, U_iou, U_f, b_iou, b_f):
    bm = 2000
    grid = (N // bm,)

    wf_x, p = pl.pallas_call(
        _pre_body,
        grid=grid,
        in_specs=[
            pl.BlockSpec((bm, H), lambda i: (i, 0)),
            pl.BlockSpec((bm, H), lambda i: (i, 0)),
            pl.BlockSpec((H, H), lambda i: (0, 0)),
            pl.BlockSpec((2 * H, H), lambda i: (0, 0)),
            pl.BlockSpec((1, H), lambda i: (0, 0)),
        ],
        out_specs=[
            pl.BlockSpec((bm, H), lambda i: (i, 0)),
            pl.BlockSpec((2, bm, H), lambda i: (0, i, 0)),
        ],
        out_shape=[
            jax.ShapeDtypeStruct((N, H), jnp.float32),
            jax.ShapeDtypeStruct((2, N, H), jnp.float32),
        ],
    )(x, h, W_f, U_f, b_f)

    src = edge_index[0]
    dst = edge_index[1]
    el = edge_label[:, 0]
    p4 = p.reshape(2 * N, 4, HQ).reshape(8 * N, HQ)
    wf4 = wf_x.reshape(4 * N, HQ)
    def _pairs(a):
        return jnp.swapaxes(a.astype(jnp.bfloat16).reshape(N, 4, 2, 16), 2, 3)
    hc4 = lax.bitcast_convert_type(
        jnp.concatenate([_pairs(h), _pairs(c)], axis=2), jnp.int32
    ).reshape(4 * N, HQ)

    t_acc, c_agg = _sc_edge()(src, dst, el, p4, hc4, wf4)

    h_new, c_new = pl.pallas_call(
        _post_body,
        grid=grid,
        in_specs=[
            pl.BlockSpec((bm, H), lambda i: (i, 0)),
            pl.BlockSpec((H, 3 * H), lambda i: (0, 0)),
            pl.BlockSpec((4, 2, bm, HQ), lambda i: (0, 0, i, 0)),
            pl.BlockSpec((2 * H, 3 * H), lambda i: (0, 0)),
            pl.BlockSpec((1, 3 * H), lambda i: (0, 0)),
            pl.BlockSpec((4, bm, HQ), lambda i: (0, i, 0)),
        ],
        out_specs=[
            pl.BlockSpec((bm, H), lambda i: (i, 0)),
            pl.BlockSpec((bm, H), lambda i: (i, 0)),
        ],
        out_shape=[
            jax.ShapeDtypeStruct((N, H), jnp.float32),
            jax.ShapeDtypeStruct((N, H), jnp.float32),
        ],
    )(x, W_iou, t_acc, U_iou, b_iou, c_agg)

    return h_new, c_new


# speedup vs baseline: 2.0489x; 1.0024x over previous
import functools

import jax
import jax.numpy as jnp
from jax import lax
from jax.experimental import pallas as pl
from jax.experimental.pallas import tpu as pltpu
from jax.experimental.pallas import tpu_sc as plsc

N = 10000
H = 128
E = 320000
NSC = 2
NSUB = 16
NPASS = 2
HQ = H // (NSC * NPASS)
ET = E // NSUB
SCE = 4000
NSUP = ET // SCE
K = 80
NCHS = SCE // K
ZR = 200
ZJOB = 2000


def _pre_body(x_ref, h_ref, wf_ref, uf_ref, bf_ref, wfx_out, p_out):
    xv = x_ref[...]
    hv = h_ref[...]
    uf = uf_ref[...]
    wfx_out[...] = jnp.dot(xv, wf_ref[...], preferred_element_type=jnp.float32) + bf_ref[...]
    p0 = jnp.dot(hv, uf[0:H, :], preferred_element_type=jnp.float32)
    p1 = jnp.dot(hv, uf[H:2 * H, :], preferred_element_type=jnp.float32)
    p_out[...] = jnp.stack([p0, p1], axis=0)


def _post_body(x_ref, wiou_ref, t_ref, u_ref, biou_ref, cagg_ref, h_out, c_out):
    xv = x_ref[...]
    tv = t_ref[...]
    uv = u_ref[...]
    s_blk = jnp.concatenate(
        [tv[cq, t] for t in range(2) for cq in range(4)], axis=1)
    iou = (jnp.dot(xv, wiou_ref[...], preferred_element_type=jnp.float32)
           + jnp.dot(s_blk, uv, preferred_element_type=jnp.float32)
           + biou_ref[...])
    i = jax.nn.sigmoid(iou[:, 0:H])
    o = jax.nn.sigmoid(iou[:, H:2 * H])
    u = jnp.tanh(iou[:, 2 * H:3 * H])
    ca = cagg_ref[...]
    c_agg = jnp.concatenate([ca[cq] for cq in range(4)], axis=1)
    c_new = i * u + c_agg
    h_out[...] = o * jnp.tanh(c_new)
    c_out[...] = c_new


def _sc_edge_body(src_hbm, dst_hbm, el_hbm, p4_hbm, hc4_hbm, wf4_hbm,
                  t_out, c_out,
                  se_s, se_d, se_l,
                  ip0, ip1, isc0, isc1, iw0, iw1, it0, it1, ic0, ic1,
                  fm0, fm1, hcs0, hcs1, wf0, wf1, fc0, fc1, hf0, hf1,
                  zb, tacc, cacc, g0, g1, s0, s1):
    cidx = lax.axis_index("c")
    sidx = lax.axis_index("s")

    ip_, isc_, iw_, it_, ic_ = (ip0, ip1), (isc0, isc1), (iw0, iw1), (it0, it1), (ic0, ic1)
    fm_, hcs_, wf_ = (fm0, fm1), (hcs0, hcs1), (wf0, wf1)
    fc_, hf_ = (fc0, fc1), (hf0, hf1)
    gsem, ssem = (g0, g1), (s0, s1)

    def _zb_row(r, _):
        for j in range(HQ // 16):
            zb[r, pl.ds(j * 16, 16)] = jnp.zeros((16,), jnp.float32)
        return _
    lax.fori_loop(0, ZR, _zb_row, None)

    for q in range(NPASS):
        cq = 2 * q + cidx

        for w in range(15):
            @pl.when(sidx == w)
            def _zero_job(w=w):
                def _z(i, _):
                    if w < 10:
                        pltpu.sync_copy(zb, tacc.at[pl.ds(w * ZJOB + i * ZR, ZR)])
                    else:
                        pltpu.sync_copy(zb, cacc.at[pl.ds((w - 10) * ZJOB + i * ZR, ZR)])
                    return _
                lax.fori_loop(0, ZJOB // ZR, _z, None)

        plsc.subcore_barrier()

        def _idx(b, o):
            for sl in range(K // 16):
                sls = pl.ds(sl * 16, 16)
                s_v = se_s[pl.ds(o + sl * 16, 16)]
                d_v = se_d[pl.ds(o + sl * 16, 16)]
                l_v = se_l[pl.ds(o + sl * 16, 16)].astype(jnp.int32)
                ip_[b][sls] = 4 * (s_v + N * l_v) + cq
                isc_[b][sls] = 4 * s_v + cq
                iw_[b][sls] = 4 * d_v + cq
                it_[b][sls] = d_v + N * l_v
                ic_[b][sls] = d_v

        KH = K // 2

        def _gather_issue(b):
            for hh in range(2):
                sh = pl.ds(hh * KH, KH)
                pltpu.async_copy(p4_hbm.at[ip_[b].at[sh]], fm_[b].at[sh], gsem[b])
                pltpu.async_copy(hc4_hbm.at[isc_[b].at[sh]], hcs_[b].at[sh], gsem[b])
                pltpu.async_copy(wf4_hbm.at[iw_[b].at[sh]], wf_[b].at[sh], gsem[b])

        def _gather_wait(b):
            for hh in range(2):
                sh = pl.ds(hh * KH, KH)
                pltpu.make_async_copy(p4_hbm.at[ip_[b].at[sh]], fm_[b].at[sh], gsem[b]).wait()
                pltpu.make_async_copy(hc4_hbm.at[isc_[b].at[sh]], hcs_[b].at[sh], gsem[b]).wait()
                pltpu.make_async_copy(wf4_hbm.at[iw_[b].at[sh]], wf_[b].at[sh], gsem[b]).wait()

        def _scatter_issue(b):
            pltpu.async_copy(fc_[b], cacc.at[ic_[b]], ssem[b], add=True)
            pltpu.async_copy(hf_[b], tacc.at[it_[b]], ssem[b], add=True)

        def _scatter_wait(b):
            pltpu.make_async_copy(fc_[b], cacc.at[ic_[b]], ssem[b]).wait()
            pltpu.make_async_copy(hf_[b], tacc.at[it_[b]], ssem[b]).wait()

        MASK_HI = jnp.int32(-65536)

        def _bf2(v):
            lo = lax.bitcast_convert_type(v << 16, jnp.float32)
            hi = lax.bitcast_convert_type(v & MASK_HI, jnp.float32)
            return lo, hi

        def _compute_f(b):
            def _f_row(e4, _3):
                for r in range(4):
                    e = e4 * 4 + r
                    ha, hb = _bf2(hcs_[b][e, pl.ds(0, 16)])
                    ca, cb = _bf2(hcs_[b][e, pl.ds(16, 16)])
                    xa = wf_[b][e, pl.ds(0, 16)] + fm_[b][e, pl.ds(0, 16)]
                    xb = wf_[b][e, pl.ds(16, 16)] + fm_[b][e, pl.ds(16, 16)]
                    fva = 1.0 / (1.0 + jnp.exp(-xa))
                    fvb = 1.0 / (1.0 + jnp.exp(-xb))
                    fc_[b][e, pl.ds(0, 16)] = fva * ca
                    fc_[b][e, pl.ds(16, 16)] = fvb * cb
                    hf_[b][e, pl.ds(0, 16)] = ha
                    hf_[b][e, pl.ds(16, 16)] = hb
                return _3
            lax.fori_loop(0, K // 4, _f_row, None)

        def _sup(s, _):
            base = sidx * ET + s * SCE
            pltpu.sync_copy(src_hbm.at[pl.ds(base, SCE)], se_s)
            pltpu.sync_copy(dst_hbm.at[pl.ds(base, SCE)], se_d)
            pltpu.sync_copy(el_hbm.at[pl.ds(base, SCE)], se_l)

            _idx(0, 0)
            _gather_issue(0)

            def _pair(j2, _2):
                for b in range(2):
                    j = j2 * 2 + b
                    b1 = 1 - b
                    _gather_wait(b)

                    @pl.when(j > 0)
                    def _w():
                        _scatter_wait(b1)

                    @pl.when(j < NCHS - 1)
                    def _n():
                        _idx(b1, (j + 1) * K)
                        _gather_issue(b1)

                    _compute_f(b)
                    _scatter_issue(b)
                return _2
            lax.fori_loop(0, NCHS // 2, _pair, None)
            _scatter_wait(1)
            return _
        lax.fori_loop(0, NSUP, _sup, None)

        plsc.subcore_barrier()

        for w in range(15):
            @pl.when(sidx == w)
            def _out_job(w=w):
                if w < 10:
                    t, i = w // 5, w % 5
                    pltpu.sync_copy(tacc.at[pl.ds(t * N + i * ZJOB, ZJOB)],
                                    t_out.at[cq, t, pl.ds(i * ZJOB, ZJOB)])
                else:
                    i = w - 10
                    pltpu.sync_copy(cacc.at[pl.ds(i * ZJOB, ZJOB)],
                                    c_out.at[cq, pl.ds(i * ZJOB, ZJOB)])

        plsc.subcore_barrier()


@functools.lru_cache(maxsize=1)
def _sc_edge():
  return pl.kernel(
    _sc_edge_body,
    mesh=plsc.VectorSubcoreMesh(core_axis_name="c", subcore_axis_name="s"),
    compiler_params=pltpu.CompilerParams(use_tc_tiling_on_sc=False),
    out_type=[
        jax.ShapeDtypeStruct((4, 2, N, HQ), jnp.float32),
        jax.ShapeDtypeStruct((4, N, HQ), jnp.float32),
    ],
    scratch_types=(
        [pltpu.VMEM((SCE,), jnp.int32),
         pltpu.VMEM((SCE,), jnp.int32),
         pltpu.VMEM((SCE,), jnp.float32)]
        + [pltpu.VMEM((K,), jnp.int32)] * 10
        + [pltpu.VMEM((K, HQ), jnp.float32)] * 2
        + [pltpu.VMEM((K, HQ), jnp.int32)] * 2
        + [pltpu.VMEM((K, HQ), jnp.float32)] * 6
        + [pltpu.VMEM((ZR, HQ), jnp.float32),
           pltpu.VMEM_SHARED((2 * N, HQ), jnp.float32),
           pltpu.VMEM_SHARED((N, HQ), jnp.float32)]
        + [pltpu.SemaphoreType.DMA] * 4
    ),
  )


def kernel(x, h, c, edge_index, edge_label, W_iou, W_f, U_iou, U_f, b_iou, b_f):
    bm = 2000
    grid = (N // bm,)

    wf_x, p = pl.pallas_call(
        _pre_body,
        grid=grid,
        in_specs=[
            pl.BlockSpec((bm, H), lambda i: (i, 0)),
            pl.BlockSpec((bm, H), lambda i: (i, 0)),
            pl.BlockSpec((H, H), lambda i: (0, 0)),
            pl.BlockSpec((2 * H, H), lambda i: (0, 0)),
            pl.BlockSpec((1, H), lambda i: (0, 0)),
        ],
        out_specs=[
            pl.BlockSpec((bm, H), lambda i: (i, 0)),
            pl.BlockSpec((2, bm, H), lambda i: (0, i, 0)),
        ],
        out_shape=[
            jax.ShapeDtypeStruct((N, H), jnp.float32),
            jax.ShapeDtypeStruct((2, N, H), jnp.float32),
        ],
    )(x, h, W_f, U_f, b_f)

    src = edge_index[0]
    dst = edge_index[1]
    el = edge_label[:, 0]
    p4 = p.reshape(2 * N, 4, HQ).reshape(8 * N, HQ)
    wf4 = wf_x.reshape(4 * N, HQ)
    def _pairs(a):
        return jnp.swapaxes(a.astype(jnp.bfloat16).reshape(N, 4, 2, 16), 2, 3)
    hc4 = lax.bitcast_convert_type(
        jnp.concatenate([_pairs(h), _pairs(c)], axis=2), jnp.int32
    ).reshape(4 * N, HQ)

    t_acc, c_agg = _sc_edge()(src, dst, el, p4, hc4, wf4)

    h_new, c_new = pl.pallas_call(
        _post_body,
        grid=grid,
        in_specs=[
            pl.BlockSpec((bm, H), lambda i: (i, 0)),
            pl.BlockSpec((H, 3 * H), lambda i: (0, 0)),
            pl.BlockSpec((4, 2, bm, HQ), lambda i: (0, 0, i, 0)),
            pl.BlockSpec((2 * H, 3 * H), lambda i: (0, 0)),
            pl.BlockSpec((1, 3 * H), lambda i: (0, 0)),
            pl.BlockSpec((4, bm, HQ), lambda i: (0, i, 0)),
        ],
        out_specs=[
            pl.BlockSpec((bm, H), lambda i: (i, 0)),
            pl.BlockSpec((bm, H), lambda i: (i, 0)),
        ],
        out_shape=[
            jax.ShapeDtypeStruct((N, H), jnp.float32),
            jax.ShapeDtypeStruct((N, H), jnp.float32),
        ],
    )(x, W_iou, t_acc, U_iou, b_iou, c_agg)

    return h_new, c_new
